# baseline (speedup 1.0000x reference)
import functools

import jax
import jax.numpy as jnp
from jax import lax
from jax.experimental import pallas as pl
from jax.experimental.pallas import tpu as pltpu
from jax.experimental.pallas import tpu_sc as plsc

B = 1024
T = 50
TP = 52
P = 20
ED = 16
TD = 16
FD = ED + TD + ED

NC = 2
NS = 16
NW = NC * NS
BW = B // NW
TEXT_PER_W = BW * TP
PREV_PER_W = BW * P
GCHUNK = 128
TCH = TEXT_PER_W // GCHUNK
PCH = PREV_PER_W // GCHUNK
PAD_ROW = 1


def _prep_body(tt_ref, tl_ref, pt_ref, plen_ref, mt_ref, mp_ref,
               rdt_ref, rdp_ref):
    tl = tl_ref[...]
    col_t = lax.broadcasted_iota(jnp.int32, (B, T), 1)
    mt_ref[:, :T] = jnp.where(col_t < tl, tt_ref[...], PAD_ROW)
    mt_ref[:, T:] = jnp.full((B, TP - T), PAD_ROW, jnp.int32)
    pl_ = plen_ref[...]
    col_p = lax.broadcasted_iota(jnp.int32, (B, P), 1)
    mp_ref[...] = jnp.where(col_p < pl_, pt_ref[...], PAD_ROW)
    rdt_ref[...] = 1.0 / jnp.maximum(tl.astype(jnp.float32), 1.0)
    rdp_ref[...] = 1.0 / jnp.maximum(pl_.astype(jnp.float32), 1.0)


def _sc_gather_pool(text_table, event_table, text_idx, prev_idx, e1):
    mesh = plsc.VectorSubcoreMesh(core_axis_name="c", subcore_axis_name="s")

    @functools.partial(
        pl.kernel,
        mesh=mesh,
        compiler_params=pltpu.CompilerParams(use_tc_tiling_on_sc=False),
        out_type=jax.ShapeDtypeStruct((B, FD), jnp.float32),
        scratch_types=[
            pltpu.VMEM((TEXT_PER_W,), jnp.int32),
            pltpu.VMEM((PREV_PER_W,), jnp.int32),
            pltpu.VMEM((BW,), jnp.int32),
            pltpu.VMEM((TEXT_PER_W, TD), jnp.float32),
            pltpu.VMEM((PREV_PER_W, ED), jnp.float32),
            pltpu.VMEM((BW, ED), jnp.float32),
            pltpu.VMEM((BW, FD), jnp.float32),
            pltpu.SemaphoreType.DMA,
        ],
    )
    def k(tt_hbm, et_hbm, ti_hbm, pi_hbm, e1_hbm, out_hbm,
          ti_v, pi_v, e1_v, rt_v, rp_v, re_v, mlp_v, sem):
        s = lax.axis_index("s")
        wid = s * NC + lax.axis_index("c")
        b0 = wid * BW
        pltpu.sync_copy(ti_hbm.at[pl.ds(b0 * TP, TEXT_PER_W)], ti_v)
        pltpu.sync_copy(pi_hbm.at[pl.ds(b0 * P, PREV_PER_W)], pi_v)
        pltpu.sync_copy(e1_hbm.at[pl.ds(b0, BW)], e1_v)

        handles = []
        for c in range(TCH):
            handles.append(pltpu.async_copy(
                tt_hbm.at[ti_v.at[pl.ds(c * GCHUNK, GCHUNK)]],
                rt_v.at[pl.ds(c * GCHUNK, GCHUNK)], sem))
        for c in range(PCH):
            handles.append(pltpu.async_copy(
                et_hbm.at[pi_v.at[pl.ds(c * GCHUNK, GCHUNK)]],
                rp_v.at[pl.ds(c * GCHUNK, GCHUNK)], sem))
        handles.append(pltpu.async_copy(et_hbm.at[e1_v], re_v, sem))
        for h in handles:
            h.wait()

        @pl.loop(0, BW, step=4)
        def _(j0):
            for dj in range(4):
                j = j0 + dj
                mlp_v[j, pl.ds(0, ED)] = re_v[j, :]

                tb = j * TP
                a = [jnp.zeros((TD,), jnp.float32) for _ in range(4)]
                for t in range(0, TP, 4):
                    for q in range(4):
                        a[q] = a[q] + rt_v[tb + t + q, :]
                mlp_v[j, pl.ds(ED, TD)] = (a[0] + a[1]) + (a[2] + a[3])

                pb = j * P
                a2 = [jnp.zeros((ED,), jnp.float32) for _ in range(4)]
                for t in range(0, P, 4):
                    for q in range(4):
                        a2[q] = a2[q] + rp_v[pb + t + q, :]
                mlp_v[j, pl.ds(ED + TD, ED)] = (a2[0] + a2[1]) + (a2[2] + a2[3])

        pltpu.sync_copy(mlp_v, out_hbm.at[pl.ds(b0, BW)])

    return k(text_table, event_table, text_idx, prev_idx, e1)


def _matmul_body(wt_ref, mlp_ref, rdt_ref, rdp_ref, b_ref, out_ref, mlp_s):
    @pl.when(pl.program_id(0) == 0)
    def _():
        m = mlp_ref[...]
        scale = jnp.concatenate(
            [jnp.ones((B, ED), jnp.float32),
             jnp.broadcast_to(rdt_ref[...], (B, TD)),
             jnp.broadcast_to(rdp_ref[...], (B, ED))], axis=1)
        mlp_s[...] = (m * scale).astype(jnp.bfloat16)

    acc = lax.dot_general(
        wt_ref[...].astype(jnp.bfloat16), mlp_s[...],
        (((0,), (1,)), ((), ())),
        preferred_element_type=jnp.float32)
    out_ref[...] = acc + b_ref[...].T


BN = 2048


def kernel(e1, e1_text_tokens, e1_text_lengths, e1prev_tokens, e1prev_lengths,
           event_table, text_table, W, b):
    EV = W.shape[0]

    mt, mp, rdt, rdp = pl.pallas_call(
        _prep_body,
        out_shape=(
            jax.ShapeDtypeStruct((B, TP), jnp.int32),
            jax.ShapeDtypeStruct((B, P), jnp.int32),
            jax.ShapeDtypeStruct((B, 1), jnp.float32),
            jax.ShapeDtypeStruct((B, 1), jnp.float32),
        ),
    )(e1_text_tokens.astype(jnp.int32),
      e1_text_lengths.reshape(B, 1).astype(jnp.int32),
      e1prev_tokens.astype(jnp.int32),
      e1prev_lengths.reshape(B, 1).astype(jnp.int32))

    mlp = _sc_gather_pool(
        text_table, event_table, mt.reshape(-1), mp.reshape(-1),
        e1.astype(jnp.int32))

    nblk = (EV + BN - 1) // BN
    logits_t = pl.pallas_call(
        _matmul_body,
        grid=(nblk,),
        in_specs=[
            pl.BlockSpec((FD, BN), lambda i: (0, i)),
            pl.BlockSpec((B, FD), lambda i: (0, 0)),
            pl.BlockSpec((B, 1), lambda i: (0, 0)),
            pl.BlockSpec((B, 1), lambda i: (0, 0)),
            pl.BlockSpec((1, BN), lambda i: (0, i)),
        ],
        out_specs=pl.BlockSpec((BN, B), lambda i: (i, 0)),
        out_shape=jax.ShapeDtypeStruct((EV, B), jnp.float32),
        scratch_shapes=[pltpu.VMEM((B, FD), jnp.bfloat16)],
        compiler_params=pltpu.CompilerParams(
            dimension_semantics=("arbitrary",)),
    )(W.T, mlp, rdt, rdp, b.reshape(1, EV))
    return logits_t.T


# speedup vs baseline: 1.6229x; 1.6229x over previous
import functools

import jax
import jax.numpy as jnp
from jax import lax
from jax.experimental import pallas as pl
from jax.experimental.pallas import tpu as pltpu
from jax.experimental.pallas import tpu_sc as plsc

B = 1024
T = 50
P = 20
ED = 16
TD = 16
FD = ED + TD + ED

NC = 2
NS = 16
NW = NC * NS
BW = B // NW
TEXT_PER_W = BW * T
PREV_PER_W = BW * P
GCHUNK = 128


def _chunks(total):
    offs = []
    o = 0
    while o < total:
        offs.append((o, min(GCHUNK, total - o)))
        o += GCHUNK
    return offs


def _sc_gather(text_table, event_table, text_idx, prev_idx, e1):
    mesh = plsc.VectorSubcoreMesh(core_axis_name="c", subcore_axis_name="s")

    @functools.partial(
        pl.kernel,
        mesh=mesh,
        compiler_params=pltpu.CompilerParams(use_tc_tiling_on_sc=False),
        out_type=(
            jax.ShapeDtypeStruct((B * T, TD), jnp.float32),
            jax.ShapeDtypeStruct((B * P, ED), jnp.float32),
            jax.ShapeDtypeStruct((B, ED), jnp.float32),
        ),
        scratch_types=[
            pltpu.VMEM((TEXT_PER_W,), jnp.int32),
            pltpu.VMEM((PREV_PER_W,), jnp.int32),
            pltpu.VMEM((BW,), jnp.int32),
            pltpu.VMEM((TEXT_PER_W, TD), jnp.float32),
            pltpu.VMEM((PREV_PER_W, ED), jnp.float32),
            pltpu.VMEM((BW, ED), jnp.float32),
            pltpu.SemaphoreType.DMA,
        ],
    )
    def k(tt_hbm, et_hbm, ti_hbm, pi_hbm, e1_hbm, out_t, out_p, out_e,
          ti_v, pi_v, e1_v, rt_v, rp_v, re_v, sem):
        wid = lax.axis_index("s") * NC + lax.axis_index("c")
        b0 = wid * BW
        pltpu.sync_copy(ti_hbm.at[pl.ds(b0 * T, TEXT_PER_W)], ti_v)
        pltpu.sync_copy(pi_hbm.at[pl.ds(b0 * P, PREV_PER_W)], pi_v)
        pltpu.sync_copy(e1_hbm.at[pl.ds(b0, BW)], e1_v)

        handles = []
        for off, sz in _chunks(TEXT_PER_W):
            handles.append(pltpu.async_copy(
                tt_hbm.at[ti_v.at[pl.ds(off, sz)]],
                rt_v.at[pl.ds(off, sz)], sem))
        for off, sz in _chunks(PREV_PER_W):
            handles.append(pltpu.async_copy(
                et_hbm.at[pi_v.at[pl.ds(off, sz)]],
                rp_v.at[pl.ds(off, sz)], sem))
        handles.append(pltpu.async_copy(et_hbm.at[e1_v], re_v, sem))
        for h in handles:
            h.wait()

        pltpu.sync_copy(rt_v, out_t.at[pl.ds(b0 * T, TEXT_PER_W)])
        pltpu.sync_copy(rp_v, out_p.at[pl.ds(b0 * P, PREV_PER_W)])
        pltpu.sync_copy(re_v, out_e.at[pl.ds(b0, BW)])

    return k(text_table, event_table, text_idx, prev_idx, e1)


def _pool(rows, lens, n, d):
    col = lax.broadcasted_iota(jnp.int32, (B, n * d), 1)
    mask = (lax.shift_right_logical(col, 4) < lens).astype(jnp.float32)
    sr = lax.broadcasted_iota(jnp.int32, (n * d, d), 0)
    sc = lax.broadcasted_iota(jnp.int32, (n * d, d), 1)
    sel = ((sr & (d - 1)) == sc).astype(jnp.float32)
    pooled = lax.dot_general(
        rows * mask, sel, (((1,), (0,)), ((), ())),
        preferred_element_type=jnp.float32)
    return pooled / jnp.maximum(lens.astype(jnp.float32), 1.0)


def _matmul_body(wt_ref, gt_ref, gp_ref, ge_ref, tl_ref, pl_ref, b_ref,
                 out_ref, mlp_s):
    @pl.when(pl.program_id(0) == 0)
    def _():
        enc_t = _pool(gt_ref[...], tl_ref[...], T, TD)
        enc_p = _pool(gp_ref[...], pl_ref[...], P, ED)
        mlp_s[...] = jnp.concatenate(
            [ge_ref[...], enc_t, enc_p], axis=1).astype(jnp.bfloat16)

    acc = lax.dot_general(
        wt_ref[...].astype(jnp.bfloat16), mlp_s[...],
        (((0,), (1,)), ((), ())),
        preferred_element_type=jnp.float32)
    out_ref[...] = acc + b_ref[...].T


BN = 2048


def kernel(e1, e1_text_tokens, e1_text_lengths, e1prev_tokens, e1prev_lengths,
           event_table, text_table, W, b):
    EV = W.shape[0]

    rt, rp, re = _sc_gather(
        text_table, event_table,
        e1_text_tokens.reshape(-1).astype(jnp.int32),
        e1prev_tokens.reshape(-1).astype(jnp.int32),
        e1.astype(jnp.int32))

    gt = rt.reshape(B, T * TD)
    gp = rp.reshape(B, P * ED)

    nblk = (EV + BN - 1) // BN
    logits_t = pl.pallas_call(
        _matmul_body,
        grid=(nblk,),
        in_specs=[
            pl.BlockSpec((FD, BN), lambda i: (0, i)),
            pl.BlockSpec((B, T * TD), lambda i: (0, 0)),
            pl.BlockSpec((B, P * ED), lambda i: (0, 0)),
            pl.BlockSpec((B, ED), lambda i: (0, 0)),
            pl.BlockSpec((B, 1), lambda i: (0, 0)),
            pl.BlockSpec((B, 1), lambda i: (0, 0)),
            pl.BlockSpec((1, BN), lambda i: (0, i)),
        ],
        out_specs=pl.BlockSpec((BN, B), lambda i: (i, 0)),
        out_shape=jax.ShapeDtypeStruct((EV, B), jnp.float32),
        scratch_shapes=[pltpu.VMEM((B, FD), jnp.bfloat16)],
        compiler_params=pltpu.CompilerParams(
            dimension_semantics=("arbitrary",)),
    )(W.T, gt, gp, re,
      e1_text_lengths.reshape(B, 1).astype(jnp.int32),
      e1prev_lengths.reshape(B, 1).astype(jnp.int32),
      b.reshape(1, EV))
    return logits_t.T


# speedup vs baseline: 1.6302x; 1.0045x over previous
import functools

import jax
import jax.numpy as jnp
from jax import lax
from jax.experimental import pallas as pl
from jax.experimental.pallas import tpu as pltpu
from jax.experimental.pallas import tpu_sc as plsc

B = 1024
T = 50
P = 20
ED = 16
TD = 16
FD = ED + TD + ED

NC = 2
NS = 16
NW = NC * NS
BW = B // NW
TEXT_PER_W = BW * T
PREV_PER_W = BW * P
GCHUNK = 128


def _chunks(total):
    offs = []
    o = 0
    while o < total:
        offs.append((o, min(GCHUNK, total - o)))
        o += GCHUNK
    return offs


def _sc_gather_one(table, idx, n_per_w, d):
    mesh = plsc.VectorSubcoreMesh(core_axis_name="c", subcore_axis_name="s")
    n = idx.shape[0]

    @functools.partial(
        pl.kernel,
        mesh=mesh,
        compiler_params=pltpu.CompilerParams(use_tc_tiling_on_sc=False),
        out_type=jax.ShapeDtypeStruct((n, d), jnp.float32),
        scratch_types=[
            pltpu.VMEM((n_per_w,), jnp.int32),
            pltpu.VMEM((n_per_w, d), jnp.float32),
            pltpu.SemaphoreType.DMA,
        ],
    )
    def k(t_hbm, i_hbm, out_hbm, i_v, r_v, sem):
        wid = lax.axis_index("s") * NC + lax.axis_index("c")
        b0 = wid * n_per_w
        pltpu.sync_copy(i_hbm.at[pl.ds(b0, n_per_w)], i_v)
        handles = []
        for off, sz in _chunks(n_per_w):
            handles.append(pltpu.async_copy(
                t_hbm.at[i_v.at[pl.ds(off, sz)]],
                r_v.at[pl.ds(off, sz)], sem))
        for h in handles:
            h.wait()
        pltpu.sync_copy(r_v, out_hbm.at[pl.ds(b0, n_per_w)])

    return k(table, idx)


def _sc_gather_event(event_table, prev_idx, e1):
    mesh = plsc.VectorSubcoreMesh(core_axis_name="c", subcore_axis_name="s")

    @functools.partial(
        pl.kernel,
        mesh=mesh,
        compiler_params=pltpu.CompilerParams(use_tc_tiling_on_sc=False),
        out_type=(
            jax.ShapeDtypeStruct((B * P, ED), jnp.float32),
            jax.ShapeDtypeStruct((B, ED), jnp.float32),
        ),
        scratch_types=[
            pltpu.VMEM((PREV_PER_W,), jnp.int32),
            pltpu.VMEM((BW,), jnp.int32),
            pltpu.VMEM((PREV_PER_W, ED), jnp.float32),
            pltpu.VMEM((BW, ED), jnp.float32),
            pltpu.SemaphoreType.DMA,
        ],
    )
    def k(et_hbm, pi_hbm, e1_hbm, out_p, out_e, pi_v, e1_v, rp_v, re_v, sem):
        wid = lax.axis_index("s") * NC + lax.axis_index("c")
        b0 = wid * BW
        pltpu.sync_copy(pi_hbm.at[pl.ds(b0 * P, PREV_PER_W)], pi_v)
        pltpu.sync_copy(e1_hbm.at[pl.ds(b0, BW)], e1_v)
        handles = []
        for off, sz in _chunks(PREV_PER_W):
            handles.append(pltpu.async_copy(
                et_hbm.at[pi_v.at[pl.ds(off, sz)]],
                rp_v.at[pl.ds(off, sz)], sem))
        handles.append(pltpu.async_copy(et_hbm.at[e1_v], re_v, sem))
        for h in handles:
            h.wait()
        pltpu.sync_copy(rp_v, out_p.at[pl.ds(b0 * P, PREV_PER_W)])
        pltpu.sync_copy(re_v, out_e.at[pl.ds(b0, BW)])

    return k(event_table, prev_idx, e1)


def _pool(rows, lens, n, d):
    col = lax.broadcasted_iota(jnp.int32, (B, n * d), 1)
    mask = (lax.shift_right_logical(col, 4) < lens).astype(jnp.float32)
    sr = lax.broadcasted_iota(jnp.int32, (n * d, d), 0)
    sc = lax.broadcasted_iota(jnp.int32, (n * d, d), 1)
    sel = ((sr & (d - 1)) == sc).astype(jnp.float32)
    pooled = lax.dot_general(
        rows * mask, sel, (((1,), (0,)), ((), ())),
        preferred_element_type=jnp.float32)
    return pooled / jnp.maximum(lens.astype(jnp.float32), 1.0)


def _matmul_body(wt_ref, gt_ref, gp_ref, ge_ref, tl_ref, pl_ref, b_ref,
                 out_ref, mlp_s):
    @pl.when(pl.program_id(0) == 0)
    def _():
        enc_t = _pool(gt_ref[...], tl_ref[...], T, TD)
        enc_p = _pool(gp_ref[...], pl_ref[...], P, ED)
        mlp_s[...] = jnp.concatenate(
            [ge_ref[...], enc_t, enc_p], axis=1).astype(jnp.bfloat16)

    acc = lax.dot_general(
        wt_ref[...].astype(jnp.bfloat16), mlp_s[...],
        (((0,), (1,)), ((), ())),
        preferred_element_type=jnp.float32)
    out_ref[...] = acc + b_ref[...].T


BN = 4096


def kernel(e1, e1_text_tokens, e1_text_lengths, e1prev_tokens, e1prev_lengths,
           event_table, text_table, W, b):
    EV = W.shape[0]

    rt = _sc_gather_one(
        text_table, e1_text_tokens.reshape(-1).astype(jnp.int32),
        TEXT_PER_W, TD)
    rp, re = _sc_gather_event(
        event_table, e1prev_tokens.reshape(-1).astype(jnp.int32),
        e1.astype(jnp.int32))

    gt = rt.reshape(B, T * TD)
    gp = rp.reshape(B, P * ED)

    nblk = (EV + BN - 1) // BN
    logits_t = pl.pallas_call(
        _matmul_body,
        grid=(nblk,),
        in_specs=[
            pl.BlockSpec((FD, BN), lambda i: (0, i)),
            pl.BlockSpec((B, T * TD), lambda i: (0, 0)),
            pl.BlockSpec((B, P * ED), lambda i: (0, 0)),
            pl.BlockSpec((B, ED), lambda i: (0, 0)),
            pl.BlockSpec((B, 1), lambda i: (0, 0)),
            pl.BlockSpec((B, 1), lambda i: (0, 0)),
            pl.BlockSpec((1, BN), lambda i: (0, i)),
        ],
        out_specs=pl.BlockSpec((BN, B), lambda i: (i, 0)),
        out_shape=jax.ShapeDtypeStruct((EV, B), jnp.float32),
        scratch_shapes=[pltpu.VMEM((B, FD), jnp.bfloat16)],
        compiler_params=pltpu.CompilerParams(
            dimension_semantics=("arbitrary",)),
    )(W.T, gt, gp, re,
      e1_text_lengths.reshape(B, 1).astype(jnp.int32),
      e1prev_lengths.reshape(B, 1).astype(jnp.int32),
      b.reshape(1, EV))
    return logits_t.T
